# baseline (speedup 1.0000x reference)
import jax
import jax.numpy as jnp
from jax import lax
from jax.experimental import pallas as pl
from jax.experimental.pallas import tpu as pltpu
from jax.experimental.pallas import tpu_sc as plsc

_NC = 2
_NS = 16
_NW = _NC * _NS
_LANES = 16
_ROWS = 8
_NBUF = 4


def _sc_add_body(x_hbm, e_hbm, o_hbm, *rest):
    xb = rest[0:_NBUF]
    eb = rest[_NBUF:2 * _NBUF]
    ob = rest[2 * _NBUF:3 * _NBUF]
    sem_in = rest[3 * _NBUF:4 * _NBUF]
    sem_out = rest[4 * _NBUF:5 * _NBUF]

    wid = lax.axis_index("s") * _NC + lax.axis_index("c")
    rows_total = x_hbm.shape[0]
    dim = x_hbm.shape[1]
    rows_per_w = rows_total // _NW
    n_chunks = rows_per_w // _ROWS
    vecs_per_row = dim // _LANES
    vecs_per_chunk = _ROWS * vecs_per_row
    row_shift = 6

    def chunk_row(ci):
        return (ci * _NW + wid) * _ROWS

    def fire_in(b, ci):
        r0 = chunk_row(ci)
        pltpu.async_copy(x_hbm.at[pl.ds(r0, _ROWS)], xb[b], sem_in[b])
        pltpu.async_copy(e_hbm.at[pl.ds(r0, _ROWS)], eb[b], sem_in[b])

    def wait_in(b, ci):
        r0 = chunk_row(ci)
        pltpu.make_async_copy(x_hbm.at[pl.ds(r0, _ROWS)], xb[b], sem_in[b]).wait()
        pltpu.make_async_copy(e_hbm.at[pl.ds(r0, _ROWS)], eb[b], sem_in[b]).wait()

    def fire_out(b, ci):
        r0 = chunk_row(ci)
        pltpu.async_copy(ob[b], o_hbm.at[pl.ds(r0, _ROWS)], sem_out[b])

    def wait_out(b, ci):
        r0 = chunk_row(ci)
        pltpu.make_async_copy(ob[b], o_hbm.at[pl.ds(r0, _ROWS)], sem_out[b]).wait()

    for b in range(_NBUF):
        fire_in(b, b)

    def outer(g, carry):
        for b in range(_NBUF):
            ci = g * _NBUF + b
            wait_in(b, ci)

            @pl.when(g > 0)
            def _():
                wait_out(b, ci - _NBUF)

            @plsc.parallel_loop(0, vecs_per_chunk, 1, unroll=8)
            def _(i):
                r = lax.shift_right_logical(i, row_shift)
                c = pl.multiple_of(
                    lax.shift_left(lax.bitwise_and(i, vecs_per_row - 1), 4),
                    _LANES,
                )
                ob[b][r, pl.ds(c, _LANES)] = (
                    xb[b][r, pl.ds(c, _LANES)] + eb[b][r, pl.ds(c, _LANES)]
                )

            fire_out(b, ci)

            @pl.when(ci + _NBUF < n_chunks)
            def _():
                fire_in(b, ci + _NBUF)

        return carry

    lax.fori_loop(0, n_chunks // _NBUF, outer, 0)

    for b in range(_NBUF):
        wait_out(b, n_chunks - _NBUF + b)


def kernel(x, emb_weight):
    seq_len, dim = x.shape
    mesh = plsc.VectorSubcoreMesh(core_axis_name="c", subcore_axis_name="s")
    sc_call = pl.kernel(
        _sc_add_body,
        out_type=jax.ShapeDtypeStruct((seq_len, dim), jnp.float32),
        mesh=mesh,
        compiler_params=pltpu.CompilerParams(use_tc_tiling_on_sc=True),
        scratch_types=(
            [pltpu.VMEM((_ROWS, 1024), jnp.float32)] * (3 * _NBUF)
            + [pltpu.SemaphoreType.DMA] * (2 * _NBUF)
        ),
    )
    return sc_call(x, emb_weight[:seq_len])


# speedup vs baseline: 1.0119x; 1.0119x over previous
import jax
import jax.numpy as jnp
from jax import lax
from jax.experimental import pallas as pl
from jax.experimental.pallas import tpu as pltpu
from jax.experimental.pallas import tpu_sc as plsc

_NC = 2
_NS = 16
_NW = _NC * _NS
_LANES = 16
_ROWS = 4
_NBUF = 8


def _sc_add_body(x_hbm, e_hbm, o_hbm, *rest):
    xb = rest[0:_NBUF]
    eb = rest[_NBUF:2 * _NBUF]
    ob = rest[2 * _NBUF:3 * _NBUF]
    sem_in = rest[3 * _NBUF:4 * _NBUF]
    sem_out = rest[4 * _NBUF:5 * _NBUF]

    wid = lax.axis_index("s") * _NC + lax.axis_index("c")
    rows_total = x_hbm.shape[0]
    dim = x_hbm.shape[1]
    rows_per_w = rows_total // _NW
    n_chunks = rows_per_w // _ROWS
    base = wid * rows_per_w
    vecs_per_row = dim // _LANES
    vecs_per_chunk = _ROWS * vecs_per_row
    row_shift = 6

    def fire_in(b, ci):
        r0 = base + ci * _ROWS
        pltpu.async_copy(x_hbm.at[pl.ds(r0, _ROWS)], xb[b], sem_in[b])
        pltpu.async_copy(e_hbm.at[pl.ds(r0, _ROWS)], eb[b], sem_in[b])

    def wait_in(b, ci):
        r0 = base + ci * _ROWS
        pltpu.make_async_copy(x_hbm.at[pl.ds(r0, _ROWS)], xb[b], sem_in[b]).wait()
        pltpu.make_async_copy(e_hbm.at[pl.ds(r0, _ROWS)], eb[b], sem_in[b]).wait()

    def fire_out(b, ci):
        r0 = base + ci * _ROWS
        pltpu.async_copy(ob[b], o_hbm.at[pl.ds(r0, _ROWS)], sem_out[b])

    def wait_out(b, ci):
        r0 = base + ci * _ROWS
        pltpu.make_async_copy(ob[b], o_hbm.at[pl.ds(r0, _ROWS)], sem_out[b]).wait()

    for b in range(_NBUF):
        fire_in(b, b)

    def outer(g, carry):
        for b in range(_NBUF):
            ci = g * _NBUF + b
            wait_in(b, ci)

            @pl.when(g > 0)
            def _():
                wait_out(b, ci - _NBUF)

            @plsc.parallel_loop(0, vecs_per_chunk, 1, unroll=8)
            def _(i):
                r = lax.shift_right_logical(i, row_shift)
                c = pl.multiple_of(
                    lax.shift_left(lax.bitwise_and(i, vecs_per_row - 1), 4),
                    _LANES,
                )
                ob[b][r, pl.ds(c, _LANES)] = (
                    xb[b][r, pl.ds(c, _LANES)] + eb[b][r, pl.ds(c, _LANES)]
                )

            fire_out(b, ci)

            @pl.when(ci + _NBUF < n_chunks)
            def _():
                fire_in(b, ci + _NBUF)

        return carry

    lax.fori_loop(0, n_chunks // _NBUF, outer, 0)

    for b in range(_NBUF):
        wait_out(b, n_chunks - _NBUF + b)


def kernel(x, emb_weight):
    seq_len, dim = x.shape
    mesh = plsc.VectorSubcoreMesh(core_axis_name="c", subcore_axis_name="s")
    sc_call = pl.kernel(
        _sc_add_body,
        out_type=jax.ShapeDtypeStruct((seq_len, dim), jnp.float32),
        mesh=mesh,
        compiler_params=pltpu.CompilerParams(use_tc_tiling_on_sc=True),
        scratch_types=(
            [pltpu.VMEM((_ROWS, 1024), jnp.float32)] * (3 * _NBUF)
            + [pltpu.SemaphoreType.DMA] * (2 * _NBUF)
        ),
    )
    return sc_call(x, emb_weight[:seq_len])
